# Initial kernel scaffold; baseline (speedup 1.0000x reference)
import jax
import jax.numpy as jnp
from jax.experimental import pallas as pl


def kernel(x, target):
    raise NotImplementedError("write your pallas kernel here")


# speedup vs baseline: 1.1262x; 1.1262x over previous
import functools

import numpy as np
import jax
import jax.numpy as jnp
from jax import lax
from jax.experimental import pallas as pl
from jax.experimental.pallas import tpu as pltpu
from jax.experimental.pallas import tpu_sc as plsc

_BATCH = 16384
_NCLS = 100
_S = 30.0
_MAXM = 0.5

_cnt = np.array([5000 - 50 * i for i in range(_NCLS)], dtype=np.float64)
_mnp = 1.0 / np.sqrt(np.sqrt(_cnt))
_mnp = _mnp * (_MAXM / np.max(_mnp))
_MVEC = np.asarray(_mnp, dtype=np.float32)

_NC, _NS, _L = 2, 16, 16
_NW = _NC * _NS
_RPW = _BATCH // _NW
_GRP = _RPW // _L
_UNROLL = 4

_LN2 = 0.6931471805599453
_LOG2E = 1.4426950408889634


def _ln(a):
    bits = lax.bitcast_convert_type(a, jnp.int32)
    e = jnp.right_shift(bits, 23) - 127
    f = lax.bitcast_convert_type(
        jnp.bitwise_or(jnp.bitwise_and(bits, 0x007FFFFF), 0x3F800000),
        jnp.float32)
    big = f > 1.5
    f = jnp.where(big, 0.5 * f, f)
    e = jnp.where(big, e + 1, e)
    r = f - 1.0
    s = r / (2.0 + r)
    w = s * s
    p = 1.0 + w * (0.3333333333 + w * (0.2 + w * 0.14285714))
    return e.astype(jnp.float32) * _LN2 + 2.0 * s * p


_mesh = plsc.VectorSubcoreMesh(core_axis_name="c", subcore_axis_name="s")


@functools.partial(
    pl.kernel,
    out_type=jax.ShapeDtypeStruct((_NW * _L,), jnp.float32),
    mesh=_mesh,
    scratch_types=[
        pltpu.VMEM((_NCLS, _RPW), jnp.float32),
        pltpu.VMEM((_RPW,), jnp.int32),
        pltpu.VMEM((_NCLS,), jnp.float32),
        pltpu.VMEM((_L,), jnp.float32),
    ],
    compiler_params=pltpu.CompilerParams(needs_layout_passes=False),
)
def _ldam_partials(xt_hbm, t_hbm, m_hbm, out_hbm, x_v, t_v, m_v, acc_v):
    wid = lax.axis_index("s") * _NC + lax.axis_index("c")
    base = wid * _RPW
    pltpu.sync_copy(xt_hbm.at[:, pl.ds(base, _RPW)], x_v)
    pltpu.sync_copy(t_hbm.at[pl.ds(base, _RPW)], t_v)
    pltpu.sync_copy(m_hbm, m_v)

    lane = lax.iota(jnp.int32, _L)

    def group(g, acc):
        r0 = g * _L
        t = t_v[pl.ds(r0, _L)]
        m = plsc.load_gather(m_v, [t])
        xt = plsc.load_gather(x_v, [t, r0 + lane])

        def pass_max(k, M):
            j = k * _UNROLL
            v0 = x_v[j, pl.ds(r0, _L)]
            v1 = x_v[j + 1, pl.ds(r0, _L)]
            v2 = x_v[j + 2, pl.ds(r0, _L)]
            v3 = x_v[j + 3, pl.ds(r0, _L)]
            return jnp.maximum(jnp.maximum(M, jnp.maximum(v0, v1)),
                               jnp.maximum(v2, v3))

        M = lax.fori_loop(0, _NCLS // _UNROLL, pass_max, xt)
        SM = _S * M

        def pass_sum(k, As):
            a0, a1 = As
            j = k * _UNROLL
            v0 = x_v[j, pl.ds(r0, _L)]
            v1 = x_v[j + 1, pl.ds(r0, _L)]
            v2 = x_v[j + 2, pl.ds(r0, _L)]
            v3 = x_v[j + 3, pl.ds(r0, _L)]
            e0 = jnp.exp(_S * v0 - SM)
            e1 = jnp.exp(_S * v1 - SM)
            e2 = jnp.exp(_S * v2 - SM)
            e3 = jnp.exp(_S * v3 - SM)
            return (a0 + e0 + e2, a1 + e1 + e3)

        zero = jnp.zeros((_L,), jnp.float32)
        a0, a1 = lax.fori_loop(0, _NCLS // _UNROLL, pass_sum, (zero, zero))
        A = a0 + a1
        et = jnp.exp(_S * xt - SM)
        em = jnp.exp(-_S * m)
        Ac = A + et * (em - 1.0)
        nll = _ln(Ac) + _S * (M - xt + m)
        return acc + nll

    acc = lax.fori_loop(0, _GRP, group, jnp.zeros((_L,), jnp.float32))
    acc_v[...] = acc * jnp.float32(1.0 / _BATCH)
    pltpu.sync_copy(acc_v, out_hbm.at[pl.ds(wid * _L, _L)])


def kernel(x, target):
    parts = _ldam_partials(x.T, target, _MVEC)
    return jnp.sum(parts)


# speedup vs baseline: 1.3972x; 1.2406x over previous
import functools

import jax
import jax.numpy as jnp
from jax import lax
from jax.experimental import pallas as pl
from jax.experimental.pallas import tpu as pltpu
from jax.experimental.pallas import tpu_sc as plsc

_BATCH = 16384
_NCLS = 100
_S = 30.0

_R_SC = 4096
_R_TC = _BATCH - _R_SC

_NC, _NS, _L = 2, 16, 16
_NW = _NC * _NS
_RPW = _R_SC // _NW
_NCHUNK = max(1, _RPW // 128)
_CW = _RPW // _NCHUNK
_GPC = _CW // _L
_U = 10

_LN2 = 0.6931471805599453

_C_TC = 4096


def _ln(a):
    bits = lax.bitcast_convert_type(a, jnp.int32)
    e = jnp.right_shift(bits, 23) - 127
    f = lax.bitcast_convert_type(
        jnp.bitwise_or(jnp.bitwise_and(bits, 0x007FFFFF), 0x3F800000),
        jnp.float32)
    big = f > 1.5
    f = jnp.where(big, 0.5 * f, f)
    e = jnp.where(big, e + 1, e)
    r = f - 1.0
    s = r / (2.0 + r)
    w = s * s
    p = 1.0 + w * (0.3333333333 + w * (0.2 + w * 0.14285714))
    return e.astype(jnp.float32) * _LN2 + 2.0 * s * p


_mesh = plsc.VectorSubcoreMesh(core_axis_name="c", subcore_axis_name="s")


@functools.partial(
    pl.kernel,
    out_type=jax.ShapeDtypeStruct((_NW * _L,), jnp.float32),
    mesh=_mesh,
    scratch_types=[
        [pltpu.VMEM((_NCLS, _CW), jnp.float32) for _ in range(_NCHUNK)],
        pltpu.VMEM((_NCLS, _L), jnp.float32),
        pltpu.VMEM((_RPW,), jnp.int32),
        pltpu.VMEM((_L,), jnp.float32),
        [pltpu.SemaphoreType.DMA for _ in range(_NCHUNK)],
        pltpu.SemaphoreType.DMA,
    ],
    compiler_params=pltpu.CompilerParams(needs_layout_passes=False),
)
def _ldam_sc(xt_hbm, t_hbm, out_hbm, x_c, sx_v, t_v, acc_v, sems, tsem):
    wid = lax.axis_index("s") * _NC + lax.axis_index("c")
    base = wid * _RPW
    th = pltpu.async_copy(t_hbm.at[pl.ds(base, _RPW)], t_v, tsem)
    handles = [
        pltpu.async_copy(
            xt_hbm.at[:, pl.ds(base + c * _CW, _CW)], x_c[c], sems[c])
        for c in range(_NCHUNK)
    ]
    th.wait()

    lane = lax.iota(jnp.int32, _L)

    def make_group(x_v, chunk_row0):
        def group(g, acc):
            r0 = g * _L
            t = t_v[pl.ds(chunk_row0 + r0, _L)]

            def pass_max(k, M2):
                j = k * _U
                vs = [x_v[j + u, pl.ds(r0, _L)] for u in range(_U)]
                for u in range(_U):
                    sv = _S * vs[u]
                    sx_v[j + u, :] = sv
                    M2 = jnp.maximum(M2, sv)
                return M2

            ninf = jnp.full((_L,), -jnp.inf, jnp.float32)
            M2 = lax.fori_loop(0, _NCLS // _U, pass_max, ninf)

            def pass_sum(k, As):
                a0, a1 = As
                j = k * _U
                es = [jnp.exp(sx_v[j + u, :] - M2) for u in range(_U)]
                p = [es[u] + es[u + 1] for u in range(0, _U, 2)]
                a0 = a0 + (p[0] + p[1])
                a1 = a1 + (p[2] + p[3])
                return (a0 + p[4], a1)

            zero = jnp.zeros((_L,), jnp.float32)
            a0, a1 = lax.fori_loop(0, _NCLS // _U, pass_sum, (zero, zero))
            A = a0 + a1

            xt2 = plsc.load_gather(sx_v, [t, lane])

            cnt = 100.0 - t.astype(jnp.float32)
            m = 0.5 * jnp.exp(-0.25 * _ln(cnt))

            et = jnp.exp(xt2 - M2)
            em = jnp.exp(-_S * m)
            Ac = A + et * (em - 1.0)
            nll = _ln(Ac) + (M2 - xt2) + _S * m
            return acc + nll
        return group

    acc = jnp.zeros((_L,), jnp.float32)
    for c in range(_NCHUNK):
        handles[c].wait()
        acc = lax.fori_loop(0, _GPC, make_group(x_c[c], c * _CW), acc)

    acc_v[...] = acc * jnp.float32(1.0 / _BATCH)
    pltpu.sync_copy(acc_v, out_hbm.at[pl.ds(wid * _L, _L)])


def _ldam_tc_body(xt_ref, t_ref, out_ref):
    @pl.when(pl.program_id(0) == 0)
    def _init():
        out_ref[...] = jnp.zeros_like(out_ref)

    xb = xt_ref[...]
    t = t_ref[...]
    iota0 = lax.broadcasted_iota(jnp.int32, (_NCLS, _C_TC), 0)
    mask = iota0 == t
    M = jnp.max(xb, axis=0, keepdims=True)
    A = jnp.sum(jnp.exp(_S * (xb - M)), axis=0, keepdims=True)
    xt = jnp.sum(jnp.where(mask, xb, 0.0), axis=0, keepdims=True)
    tf = t.astype(jnp.float32)
    m = 0.5 * jnp.exp(-0.25 * jnp.log(100.0 - tf))
    et = jnp.exp(_S * (xt - M))
    em = jnp.exp(-_S * m)
    Ac = A + et * (em - 1.0)
    nll = jnp.log(Ac) + _S * (M - xt + m)
    out_ref[...] = out_ref[...] + jnp.sum(
        nll * jnp.float32(1.0 / _BATCH), axis=1, keepdims=True)


def _ldam_tc(xt, target2d):
    grid = (_R_TC // _C_TC,)
    blk0 = _R_SC // _C_TC
    return pl.pallas_call(
        _ldam_tc_body,
        grid=grid,
        in_specs=[
            pl.BlockSpec((_NCLS, _C_TC), lambda i: (0, blk0 + i)),
            pl.BlockSpec((1, _C_TC), lambda i: (0, blk0 + i)),
        ],
        out_specs=pl.BlockSpec((1, 1), lambda i: (0, 0)),
        out_shape=jax.ShapeDtypeStruct((1, 1), jnp.float32),
    )(xt, target2d)


def kernel(x, target):
    xt = x.T
    sc_parts = _ldam_sc(xt, target)
    tc_part = _ldam_tc(xt, target.reshape(1, _BATCH))
    return jnp.sum(sc_parts) + tc_part[0, 0]
